# Initial kernel scaffold; baseline (speedup 1.0000x reference)
import jax
import jax.numpy as jnp
from jax.experimental import pallas as pl


def kernel(x, label, group_offsets, group_sizes, cid_groups, parents):
    raise NotImplementedError("write your pallas kernel here")


# speedup vs baseline: 18.4252x; 18.4252x over previous
import functools

import jax
import jax.numpy as jnp
from jax import lax
from jax.experimental import pallas as pl
from jax.experimental.pallas import tpu as pltpu


def _body(x_ref, lbl_ref, out_ref, *, nc, ch, n_nodes, hw, tiny):
    b = pl.program_id(0)
    xb = x_ref[0]
    m = jnp.max(xb, axis=0, keepdims=True)
    e = jnp.exp(xb - m)

    n = lbl_ref[0]
    is_fine = n >= nc
    nf = jnp.where(is_fine, n - nc, 0)
    g = nf // ch
    cidx = jnp.where(is_fine, g, n)

    ic = lax.broadcasted_iota(jnp.int32, (n_nodes, hw), 0)
    in_coarse = ic < nc
    s_c = jnp.sum(jnp.where(in_coarse, e, 0.0), axis=0, keepdims=True)
    e_c = jnp.sum(jnp.where(ic == cidx, e, 0.0), axis=0, keepdims=True)
    term = -jnp.log(jnp.maximum(e_c / s_c, tiny))

    grp = (ic - nc) // ch
    sel_g = jnp.logical_not(in_coarse) & (grp == g)
    s_f = jnp.sum(jnp.where(sel_g, e, 0.0), axis=0, keepdims=True)
    e_n = jnp.sum(jnp.where(ic == n, e, 0.0), axis=0, keepdims=True)
    p_f = e_n / jnp.maximum(s_f, tiny)
    term = term + jnp.where(is_fine, -jnp.log(jnp.maximum(p_f, tiny)), 0.0)

    @pl.when(b == 0)
    def _():
        out_ref[...] = jnp.zeros_like(out_ref)

    out_ref[...] += jnp.sum(term, axis=1, keepdims=True)


def kernel(x, label, group_offsets, group_sizes, cid_groups, parents):
    B, N, H, W = x.shape
    G = group_offsets.shape[0]
    nc = G - 1
    ch = (N - nc) // nc
    hw = H * W
    tiny = float(jnp.finfo(x.dtype).tiny)

    x3 = x.reshape(B, N, hw)
    lbl3 = label.reshape(B, 1, hw).astype(jnp.int32)

    body = functools.partial(_body, nc=nc, ch=ch, n_nodes=N, hw=hw, tiny=tiny)
    out = pl.pallas_call(
        body,
        grid=(B,),
        in_specs=[
            pl.BlockSpec((1, N, hw), lambda b: (b, 0, 0)),
            pl.BlockSpec((1, 1, hw), lambda b: (b, 0, 0)),
        ],
        out_specs=pl.BlockSpec((1, 1), lambda b: (0, 0)),
        out_shape=jax.ShapeDtypeStruct((1, 1), jnp.float32),
        compiler_params=pltpu.CompilerParams(
            dimension_semantics=("arbitrary",)),
    )(x3, lbl3)
    return out[0, 0] / (B * hw)
